# baseline (speedup 1.0000x reference)
import functools

import jax
import jax.numpy as jnp
import numpy as np
from jax import lax
from jax.experimental import pallas as pl
from jax.experimental.pallas import tpu as pltpu
from jax.experimental.pallas import tpu_sc as plsc

N = 10000
E = 160000
HID = 512
HEADS = 6
EMB = 64
NH = 4
HD = EMB // NH
FF = 256
TOUT = 16
NUM_LOOP = 3

SLEN = 625
SLEN_PAD = 640
SELU_SCALE = 1.0507009873554805
SELU_ALPHA = 1.6732632423543772


def _selu(x):
    return SELU_SCALE * jnp.where(x > 0, x, SELU_ALPHA * (jnp.exp(x) - 1.0))


def _layer_norm(h, g, b):
    m = jnp.mean(h, axis=-1, keepdims=True)
    v = jnp.var(h, axis=-1, keepdims=True)
    return (h - m) / jnp.sqrt(v + 1e-5) * g + b


def _transformer_body(feat_ref, W_embed, b_embed, Wq, bq, Wk, bk, Wv, bv,
                      Wo, bo, ln1_g, ln1_b, W_ff1, b_ff1, W_ff2, b_ff2,
                      ln2_g, ln2_b, W_tout, b_tout, out_ref):
    feat = feat_ref[0]
    h = feat @ W_embed[...] + b_embed[...]
    key_pad = lax.broadcasted_iota(jnp.int32, (SLEN_PAD, SLEN_PAD), 1) >= SLEN
    heads = []
    for hh in range(NH):
        sl = slice(hh * HD, (hh + 1) * HD)
        q = h @ Wq[:, sl] + bq[sl]
        k = h @ Wk[:, sl] + bk[sl]
        v = h @ Wv[:, sl] + bv[sl]
        s = lax.dot_general(q, k, (((1,), (1,)), ((), ()))) * (1.0 / np.sqrt(HD))
        s = jnp.where(key_pad, -1e30, s)
        s = s - jnp.max(s, axis=-1, keepdims=True)
        es = jnp.exp(s)
        attn = es / jnp.sum(es, axis=-1, keepdims=True)
        heads.append(attn @ v)
    o = jnp.concatenate(heads, axis=1)
    h = _layer_norm(h + (o @ Wo[...] + bo[...]), ln1_g[...], ln1_b[...])
    f = jnp.maximum(h @ W_ff1[...] + b_ff1[...], 0.0) @ W_ff2[...] + b_ff2[...]
    h = _layer_norm(h + f, ln2_g[...], ln2_b[...])
    out_ref[0] = h @ W_tout[...] + b_tout[...]


def _transformer_tc(mesh_feat, p):
    feat = mesh_feat.reshape(16, SLEN, 4)
    feat = jnp.pad(feat, ((0, 0), (0, SLEN_PAD - SLEN), (0, 0)))
    wnames = ['W_embed', 'b_embed', 'Wq', 'bq', 'Wk', 'bk', 'Wv', 'bv',
              'Wo', 'bo', 'ln1_g', 'ln1_b', 'W_ff1', 'b_ff1', 'W_ff2',
              'b_ff2', 'ln2_g', 'ln2_b', 'W_tout', 'b_tout']
    ws = [p[n] for n in wnames]
    out = pl.pallas_call(
        _transformer_body,
        grid=(16,),
        in_specs=[pl.BlockSpec((1, SLEN_PAD, 4), lambda b: (b, 0, 0))]
                 + [pl.BlockSpec(w.shape, lambda b, nd=w.ndim: (0,) * nd)
                    for w in ws],
        out_specs=pl.BlockSpec((1, SLEN_PAD, TOUT), lambda b: (b, 0, 0)),
        out_shape=jax.ShapeDtypeStruct((16, SLEN_PAD, TOUT), jnp.float32),
    )(feat, *ws)
    return out[:, :SLEN, :].reshape(N, TOUT)


def _hidden_body(x2_ref, tout_ref, W_lin, b_lin, out_ref):
    acc = x2_ref[...] @ W_lin[:5, :] + tout_ref[...] @ W_lin[5:, :]
    out_ref[...] = _selu(acc + b_lin[...])


def _hidden_tc(x2, tout, W_lin, b_lin):
    blk = 2000
    return pl.pallas_call(
        _hidden_body,
        grid=(N // blk,),
        in_specs=[
            pl.BlockSpec((blk, 5), lambda i: (i, 0)),
            pl.BlockSpec((blk, TOUT), lambda i: (i, 0)),
            pl.BlockSpec(W_lin.shape, lambda i: (0, 0)),
            pl.BlockSpec(b_lin.shape, lambda i: (0,)),
        ],
        out_specs=pl.BlockSpec((blk, HID), lambda i: (i, 0)),
        out_shape=jax.ShapeDtypeStruct((N, HID), jnp.float32),
    )(x2, tout, W_lin, b_lin)


def _xlxr_body(coord_ref, hidden_ref, Wl, Wr, xl_ref, xr_ref):
    c = coord_ref[...]
    hdn = hidden_ref[...]
    xl_ref[...] = c @ Wl[:2, :] + hdn @ Wl[2:, :]
    xr_ref[...] = c @ Wr[:2, :] + hdn @ Wr[2:, :]


def _xlxr_tc(coord, hidden, Wl, Wr):
    blk = 1000
    return pl.pallas_call(
        _xlxr_body,
        grid=(N // blk, HEADS),
        in_specs=[
            pl.BlockSpec((blk, 2), lambda i, h: (i, 0)),
            pl.BlockSpec((blk, HID), lambda i, h: (i, 0)),
            pl.BlockSpec((2 + HID, HID), lambda i, h: (0, h)),
            pl.BlockSpec((2 + HID, HID), lambda i, h: (0, h)),
        ],
        out_specs=[
            pl.BlockSpec((blk, HID), lambda i, h: (i, h)),
            pl.BlockSpec((blk, HID), lambda i, h: (i, h)),
        ],
        out_shape=[
            jax.ShapeDtypeStruct((N, HEADS * HID), jnp.float32),
            jax.ShapeDtypeStruct((N, HEADS * HID), jnp.float32),
        ],
    )(coord, hidden, Wl, Wr)


def _combine_body(agg_ref, coord_ref, gat_bias, W_coord, b_coord,
                  coord_out_ref, hid_ref):
    aggsum = jnp.sum(agg_ref[...], axis=0) * (1.0 / HEADS)
    hid = _selu(aggsum + gat_bias[...])
    oc = hid @ W_coord[...] + b_coord[...]
    coord = coord_ref[...]
    up = coord[:, 0:1] == 1.0
    down = coord[:, 0:1] == 0.0
    left = coord[:, 1:2] == 0.0
    right = coord[:, 1:2] == 1.0
    oc0 = jnp.where(down, 0.0, jnp.where(up, 1.0, oc[:, 0:1]))
    oc1 = jnp.where(right, 1.0, jnp.where(left, 0.0, oc[:, 1:2]))
    coord_out_ref[...] = jnp.concatenate([oc0, oc1], axis=1)
    hid_ref[...] = hid


def _combine_tc(agg, coord, gat_bias, W_coord, b_coord):
    blk = 1000
    return pl.pallas_call(
        _combine_body,
        grid=(N // blk,),
        in_specs=[
            pl.BlockSpec((HEADS, blk, HID), lambda i: (0, i, 0)),
            pl.BlockSpec((blk, 2), lambda i: (i, 0)),
            pl.BlockSpec(gat_bias.shape, lambda i: (0,)),
            pl.BlockSpec(W_coord.shape, lambda i: (0, 0)),
            pl.BlockSpec(b_coord.shape, lambda i: (0,)),
        ],
        out_specs=[
            pl.BlockSpec((blk, 2), lambda i: (i, 0)),
            pl.BlockSpec((blk, HID), lambda i: (i, 0)),
        ],
        out_shape=[
            jax.ShapeDtypeStruct((N, 2), jnp.float32),
            jax.ShapeDtypeStruct((N, HID), jnp.float32),
        ],
    )(agg, coord, gat_bias, W_coord, b_coord)


def _gat_edges(xl, xr, src, dst, att):
    aggs = []
    for hh in range(HEADS):
        xl_h = xl[:, hh * HID:(hh + 1) * HID]
        xr_h = xr[:, hh * HID:(hh + 1) * HID]
        e = jax.nn.leaky_relu(xl_h[src] + xr_h[dst], negative_slope=0.2)
        alpha = e @ att[hh]
        amax = jax.ops.segment_max(alpha, dst, num_segments=N)
        amax = jnp.where(jnp.isfinite(amax), amax, 0.0)
        ex = jnp.exp(alpha - amax[dst])
        denom = jax.ops.segment_sum(ex, dst, num_segments=N)
        w = ex / (denom[dst] + 1e-16)
        aggs.append(jax.ops.segment_sum(w[:, None] * xl_h[src], dst,
                                        num_segments=N))
    return jnp.stack(aggs, axis=0)


def kernel(x, edge_index, mesh_feat, conv_feat, W_embed, b_embed, Wq, bq, Wk, bk,
           Wv, bv, Wo, bo, ln1_g, ln1_b, W_ff1, b_ff1, W_ff2, b_ff2, ln2_g, ln2_b,
           W_tout, b_tout, W_lin, b_lin, Wl, Wr, att, gat_bias, W_coord, b_coord):
    p = {
        'W_embed': W_embed, 'b_embed': b_embed, 'Wq': Wq, 'bq': bq, 'Wk': Wk,
        'bk': bk, 'Wv': Wv, 'bv': bv, 'Wo': Wo, 'bo': bo, 'ln1_g': ln1_g,
        'ln1_b': ln1_b, 'W_ff1': W_ff1, 'b_ff1': b_ff1, 'W_ff2': W_ff2,
        'b_ff2': b_ff2, 'ln2_g': ln2_g, 'ln2_b': ln2_b, 'W_tout': W_tout,
        'b_tout': b_tout,
    }
    coord = x[:, :2]
    src, dst = edge_index[0], edge_index[1]

    order = jnp.argsort(dst)
    src_s = src[order]
    dst_s = dst[order]
    rowptr = jnp.searchsorted(dst_s, jnp.arange(N + 1, dtype=jnp.int32),
                              method='scan_unrolled').astype(jnp.int32)

    tout = _transformer_tc(mesh_feat, p)
    hidden = _hidden_tc(x[:, 2:], tout, W_lin, b_lin)
    for _ in range(NUM_LOOP):
        xl, xr = _xlxr_tc(coord, hidden, Wl, Wr)
        agg = _gat_edges(xl, xr, src_s, dst_s, att)
        coord, hidden = _combine_tc(agg, coord, gat_bias, W_coord, b_coord)
    return coord
